# Initial kernel scaffold; baseline (speedup 1.0000x reference)
import functools

import jax
import jax.numpy as jnp
from jax import lax
from jax.experimental import pallas as pl
from jax.experimental.pallas import tpu as pltpu
from jax.experimental.pallas import tpu_sc as plsc

N_NODES = 10000
N_EDGES = 320000
D = 128

NC = 2
NS = 16
NW = NC * NS
EPW = N_EDGES // NW
C = 80
NCHUNK = EPW // C
NP = 10240
RPT = NP // NS
LANES = 16

_mesh = plsc.VectorSubcoreMesh(core_axis_name="c", subcore_axis_name="s")


@functools.partial(
    pl.kernel,
    out_type=jax.ShapeDtypeStruct((NC * NP,), jnp.float32),
    mesh=_mesh,
    scratch_types=[
        pltpu.VMEM((C,), jnp.int32),
        pltpu.VMEM((C,), jnp.float32),
        pltpu.VMEM((RPT,), jnp.float32),
        pltpu.VMEM_SHARED((NP,), jnp.float32),
    ],
)
def _sc_degree(dst_hbm, ew_hbm, out_hbm, dstv, ewv, zbuf, degsh):
    c = lax.axis_index("c")
    s = lax.axis_index("s")
    wid = s * NC + c

    zeros16 = jnp.zeros((LANES,), jnp.float32)
    def zero_body(i, _):
        zbuf[pl.ds(i * LANES, LANES)] = zeros16
        return _
    lax.fori_loop(0, RPT // LANES, zero_body, None)
    pltpu.sync_copy(zbuf, degsh.at[pl.ds(s * RPT, RPT)])
    plsc.subcore_barrier()

    base = wid * EPW

    def chunk_body(i, _):
        off = base + i * C
        pltpu.sync_copy(dst_hbm.at[pl.ds(off, C)], dstv)
        pltpu.sync_copy(ew_hbm.at[pl.ds(off, C)], ewv)
        for j in range(C // LANES):
            sl = pl.ds(j * LANES, LANES)
            ewv[sl] = (ewv[sl] + 1.0) * 0.5
        pltpu.sync_copy(ewv, degsh.at[dstv], add=True)
        return _

    lax.fori_loop(0, NCHUNK, chunk_body, None)
    plsc.subcore_barrier()
    pltpu.sync_copy(degsh.at[pl.ds(s * RPT, RPT)],
                    out_hbm.at[pl.ds(c * NP + s * RPT, RPT)])


@functools.partial(
    pl.kernel,
    out_type=jax.ShapeDtypeStruct((NC * NP, D), jnp.float32),
    mesh=_mesh,
    scratch_types=[
        pltpu.VMEM((C,), jnp.int32),
        pltpu.VMEM((C,), jnp.int32),
        pltpu.VMEM((C,), jnp.float32),
        pltpu.VMEM((C, D), jnp.float32),
        pltpu.VMEM_SHARED((NP, D), jnp.float32),
        pltpu.SemaphoreType.DMA,
    ],
)
def _sc_aggregate(y_hbm, src_hbm, dst_hbm, ew_hbm, out_hbm,
                  srcv, dstv, ewv, rows, accsh, sem):
    c = lax.axis_index("c")
    s = lax.axis_index("s")
    wid = s * NC + c

    zeros16 = jnp.zeros((LANES,), jnp.float32)
    def zrow(e, _):
        for j in range(D // LANES):
            rows[e, pl.ds(j * LANES, LANES)] = zeros16
        return _
    lax.fori_loop(0, C, zrow, None)
    for k in range(RPT // C):
        pltpu.sync_copy(rows, accsh.at[pl.ds(s * RPT + k * C, C)])
    plsc.subcore_barrier()

    base = wid * EPW

    def chunk_body(i, _):
        off = base + i * C
        pltpu.sync_copy(src_hbm.at[pl.ds(off, C)], srcv)
        pltpu.sync_copy(dst_hbm.at[pl.ds(off, C)], dstv)
        pltpu.sync_copy(ew_hbm.at[pl.ds(off, C)], ewv)
        pltpu.async_copy(y_hbm.at[srcv], rows, sem).wait()

        def edge_body(e, _):
            w = (ewv[e] + 1.0) * 0.5
            for j in range(D // LANES):
                sl = pl.ds(j * LANES, LANES)
                rows[e, sl] = rows[e, sl] * w
            return _
        lax.fori_loop(0, C, edge_body, None)

        pltpu.sync_copy(rows, accsh.at[dstv], add=True)
        return _

    lax.fori_loop(0, NCHUNK, chunk_body, None)
    plsc.subcore_barrier()
    pltpu.sync_copy(accsh.at[pl.ds(s * RPT, RPT)],
                    out_hbm.at[pl.ds(c * NP + s * RPT, RPT)])


_NB = 10
_BN = N_NODES // _NB


def _tc_linear_body(x_ref, w_ref, degp_ref, y_ref):
    xw = jnp.dot(x_ref[...], w_ref[...], preferred_element_type=jnp.float32)
    deg = 1.0 + degp_ref[0, :] + degp_ref[1, :]
    dinv = lax.rsqrt(deg)
    y_ref[...] = xw * dinv[:, None]


def _tc_post_body(a0_ref, a1_ref, y_ref, degp_ref, b_ref, g_ref, be_ref, o_ref):
    deg = 1.0 + degp_ref[0, :] + degp_ref[1, :]
    dinv = lax.rsqrt(deg)[:, None]
    t = dinv * (a0_ref[...] + a1_ref[...] + y_ref[...]) + b_ref[...]
    t = jnp.where(t >= 0, t, 0.01 * t)
    mu = jnp.mean(t, axis=-1, keepdims=True)
    var = jnp.mean((t - mu) ** 2, axis=-1, keepdims=True)
    o_ref[...] = (t - mu) * lax.rsqrt(var + 1e-5) * g_ref[...] + be_ref[...]


def kernel(x, edge_index, edge_weight, W, b, gamma, beta):
    src = edge_index[0].astype(jnp.int32)
    dst = edge_index[1].astype(jnp.int32)
    ew = edge_weight.astype(jnp.float32)

    degp = _sc_degree(dst, ew)
    degp2 = jnp.reshape(degp, (NC, NP))[:, :N_NODES]

    row_spec = pl.BlockSpec((_BN, D), lambda i: (i, 0))
    deg_spec = pl.BlockSpec((NC, _BN), lambda i: (0, i))
    vec_spec = pl.BlockSpec((1, D), lambda i: (0, 0))

    y = pl.pallas_call(
        _tc_linear_body,
        grid=(_NB,),
        in_specs=[row_spec, pl.BlockSpec((D, D), lambda i: (0, 0)), deg_spec],
        out_specs=row_spec,
        out_shape=jax.ShapeDtypeStruct((N_NODES, D), jnp.float32),
    )(x, W, degp2)

    acc = _sc_aggregate(y, src, dst, ew)
    acc0 = lax.slice(acc, (0, 0), (N_NODES, D))
    acc1 = lax.slice(acc, (NP, 0), (NP + N_NODES, D))

    out = pl.pallas_call(
        _tc_post_body,
        grid=(_NB,),
        in_specs=[row_spec, row_spec, row_spec, deg_spec,
                  vec_spec, vec_spec, vec_spec],
        out_specs=row_spec,
        out_shape=jax.ShapeDtypeStruct((N_NODES, D), jnp.float32),
    )(acc0, acc1, y, degp2,
      b.reshape(1, D), gamma.reshape(1, D), beta.reshape(1, D))
    return out


# speedup vs baseline: 13.0216x; 13.0216x over previous
import functools

import jax
import jax.numpy as jnp
from jax import lax
from jax.experimental import pallas as pl
from jax.experimental.pallas import tpu as pltpu
from jax.experimental.pallas import tpu_sc as plsc

N_NODES = 10000
N_EDGES = 320000
D = 128

NC = 2
NS = 16
NW = NC * NS
EPW = N_EDGES // NW
C = 80
NCHUNK = EPW // C
NP = 10240
RPT = NP // NS
LANES = 16

_mesh = plsc.VectorSubcoreMesh(core_axis_name="c", subcore_axis_name="s")


@functools.partial(
    pl.kernel,
    out_type=jax.ShapeDtypeStruct((NC * NP,), jnp.float32),
    mesh=_mesh,
    scratch_types=[
        pltpu.VMEM((C,), jnp.int32),
        pltpu.VMEM((C,), jnp.float32),
        pltpu.VMEM((RPT,), jnp.float32),
        pltpu.VMEM_SHARED((NP,), jnp.float32),
    ],
)
def _sc_degree(dst_hbm, ew_hbm, out_hbm, dstv, ewv, zbuf, degsh):
    c = lax.axis_index("c")
    s = lax.axis_index("s")
    wid = s * NC + c

    zeros16 = jnp.zeros((LANES,), jnp.float32)
    def zero_body(i, _):
        zbuf[pl.ds(i * LANES, LANES)] = zeros16
        return _
    lax.fori_loop(0, RPT // LANES, zero_body, None)
    pltpu.sync_copy(zbuf, degsh.at[pl.ds(s * RPT, RPT)])
    plsc.subcore_barrier()

    base = wid * EPW

    def chunk_body(i, _):
        off = base + i * C
        pltpu.sync_copy(dst_hbm.at[pl.ds(off, C)], dstv)
        pltpu.sync_copy(ew_hbm.at[pl.ds(off, C)], ewv)
        for j in range(C // LANES):
            sl = pl.ds(j * LANES, LANES)
            ewv[sl] = (ewv[sl] + 1.0) * 0.5
        pltpu.sync_copy(ewv, degsh.at[dstv], add=True)
        return _

    lax.fori_loop(0, NCHUNK, chunk_body, None)
    plsc.subcore_barrier()
    pltpu.sync_copy(degsh.at[pl.ds(s * RPT, RPT)],
                    out_hbm.at[pl.ds(c * NP + s * RPT, RPT)])


@functools.partial(
    pl.kernel,
    out_type=jax.ShapeDtypeStruct((NC * NP, D), jnp.float32),
    mesh=_mesh,
    scratch_types=[
        pltpu.VMEM((C,), jnp.int32),
        pltpu.VMEM((C,), jnp.int32),
        pltpu.VMEM((C,), jnp.float32),
        pltpu.VMEM((C, D), jnp.float32),
        pltpu.VMEM_SHARED((NP, D), jnp.float32),
        pltpu.SemaphoreType.DMA,
    ],
)
def _sc_aggregate(y_hbm, src_hbm, dst_hbm, ew_hbm, out_hbm,
                  srcv, dstv, ewv, rows, accsh, sem):
    c = lax.axis_index("c")
    s = lax.axis_index("s")
    wid = s * NC + c

    zeros16 = jnp.zeros((LANES,), jnp.float32)
    def zrow(e, _):
        for j in range(D // LANES):
            rows[e, pl.ds(j * LANES, LANES)] = zeros16
        return _
    lax.fori_loop(0, C, zrow, None)
    for k in range(RPT // C):
        pltpu.sync_copy(rows, accsh.at[pl.ds(s * RPT + k * C, C)])
    plsc.subcore_barrier()

    base = wid * EPW

    def chunk_body(i, _):
        off = base + i * C
        pltpu.sync_copy(src_hbm.at[pl.ds(off, C)], srcv)
        pltpu.sync_copy(dst_hbm.at[pl.ds(off, C)], dstv)
        pltpu.sync_copy(ew_hbm.at[pl.ds(off, C)], ewv)
        pltpu.async_copy(y_hbm.at[srcv], rows, sem).wait()

        def group_body(g, _):
            wv = (ewv[pl.ds(g * LANES, LANES)] + 1.0) * 0.5
            e0 = g * LANES
            for l in range(LANES):
                w = wv[l]
                for j in range(D // LANES):
                    sl = pl.ds(j * LANES, LANES)
                    rows[e0 + l, sl] = rows[e0 + l, sl] * w
            return _
        lax.fori_loop(0, C // LANES, group_body, None)

        pltpu.sync_copy(rows, accsh.at[dstv], add=True)
        return _

    lax.fori_loop(0, NCHUNK, chunk_body, None)
    plsc.subcore_barrier()
    pltpu.sync_copy(accsh.at[pl.ds(s * RPT, RPT)],
                    out_hbm.at[pl.ds(c * NP + s * RPT, RPT)])


_NB = 10
_BN = N_NODES // _NB


def _tc_linear_body(x_ref, w_ref, degp_ref, y_ref):
    xw = jnp.dot(x_ref[...], w_ref[...], preferred_element_type=jnp.float32)
    deg = 1.0 + degp_ref[:, 0] + degp_ref[:, 1]
    dinv = lax.rsqrt(deg)
    y_ref[...] = xw * dinv[:, None]


def _tc_post_body(a0_ref, a1_ref, y_ref, degp_ref, b_ref, g_ref, be_ref, o_ref):
    deg = 1.0 + degp_ref[:, 0] + degp_ref[:, 1]
    dinv = lax.rsqrt(deg)[:, None]
    t = dinv * (a0_ref[...] + a1_ref[...] + y_ref[...]) + b_ref[...]
    t = jnp.where(t >= 0, t, 0.01 * t)
    mu = jnp.mean(t, axis=-1, keepdims=True)
    var = jnp.mean((t - mu) ** 2, axis=-1, keepdims=True)
    o_ref[...] = (t - mu) * lax.rsqrt(var + 1e-5) * g_ref[...] + be_ref[...]


def kernel(x, edge_index, edge_weight, W, b, gamma, beta):
    src = edge_index[0].astype(jnp.int32)
    dst = edge_index[1].astype(jnp.int32)
    ew = edge_weight.astype(jnp.float32)

    degp = _sc_degree(dst, ew)
    degp2 = jnp.reshape(degp, (NC, NP))[:, :N_NODES].T

    row_spec = pl.BlockSpec((_BN, D), lambda i: (i, 0))
    deg_spec = pl.BlockSpec((_BN, NC), lambda i: (i, 0))
    vec_spec = pl.BlockSpec((1, D), lambda i: (0, 0))

    y = pl.pallas_call(
        _tc_linear_body,
        grid=(_NB,),
        in_specs=[row_spec, pl.BlockSpec((D, D), lambda i: (0, 0)), deg_spec],
        out_specs=row_spec,
        out_shape=jax.ShapeDtypeStruct((N_NODES, D), jnp.float32),
    )(x, W, degp2)

    acc = _sc_aggregate(y, src, dst, ew)
    acc0 = lax.slice(acc, (0, 0), (N_NODES, D))
    acc1 = lax.slice(acc, (NP, 0), (NP + N_NODES, D))

    out = pl.pallas_call(
        _tc_post_body,
        grid=(_NB,),
        in_specs=[row_spec, row_spec, row_spec, deg_spec,
                  vec_spec, vec_spec, vec_spec],
        out_specs=row_spec,
        out_shape=jax.ShapeDtypeStruct((N_NODES, D), jnp.float32),
    )(acc0, acc1, y, degp2,
      b.reshape(1, D), gamma.reshape(1, D), beta.reshape(1, D))
    return out
